# Initial kernel scaffold; baseline (speedup 1.0000x reference)
import jax
import jax.numpy as jnp
from jax.experimental import pallas as pl


def kernel(conv_output, e1, e2):
    raise NotImplementedError("write your pallas kernel here")


# speedup vs baseline: 1.7531x; 1.7531x over previous
import functools

import jax
import jax.numpy as jnp
from jax import lax
from jax.experimental import pallas as pl
from jax.experimental.pallas import tpu as pltpu
from jax.experimental.pallas import tpu_sc as plsc

B, S, F = 1024, 512, 128
NW = 32
EPW = B // NW
HALF = S // 2
NV = F // 16
NEG = -1e30

_mesh = plsc.VectorSubcoreMesh(
    core_axis_name="c", subcore_axis_name="s", num_cores=2, num_subcores=16
)


def _row_loop(buf, h, lo, hi, acc):
    def body(r, acc):
        return tuple(
            jnp.maximum(acc[v], buf[h, r, pl.ds(v * 16, 16)]) for v in range(NV)
        )

    return lax.fori_loop(lo, hi, body, acc)


@functools.partial(
    pl.kernel,
    out_type=jax.ShapeDtypeStruct((B, 3 * F), jnp.float32),
    mesh=_mesh,
    scratch_types=[
        pltpu.VMEM((2, HALF, F), jnp.float32),
        pltpu.VMEM((EPW, 3 * F), jnp.float32),
        pltpu.VMEM((EPW, 16), jnp.int32),
        pltpu.SemaphoreType.DMA,
        pltpu.SemaphoreType.DMA,
    ],
)
def _sc_piecewise_max(conv_hbm, ee_hbm, out_hbm, buf, out_v, e_v, sem0, sem1):
    wid = lax.axis_index("c") * 16 + lax.axis_index("s")
    base = wid * EPW

    pltpu.sync_copy(ee_hbm.at[pl.ds(base, EPW)], e_v)

    sems = (sem0, sem1)

    def dma(ex, h):
        return pltpu.make_async_copy(
            conv_hbm.at[base + ex, pl.ds(h * HALF, HALF)], buf.at[h], sems[h]
        )

    dma(0, 0).start()
    dma(0, 1).start()

    def ex_body(i, carry):
        evec = e_v[i]
        e1s = evec[0]
        e2s = evec[1]
        accs = [
            tuple(jnp.full((16,), NEG, jnp.float32) for _ in range(NV))
            for _ in range(3)
        ]
        for h in range(2):
            c0 = h * HALF
            dma(i, h).wait()
            a = jnp.clip(e1s + 1 - c0, 0, HALF)
            b = jnp.clip(e2s + 1 - c0, 0, HALF)
            accs[0] = _row_loop(buf, h, 0, a, accs[0])
            accs[1] = _row_loop(buf, h, a, b, accs[1])
            accs[2] = _row_loop(buf, h, b, HALF, accs[2])

            @pl.when(i + 1 < EPW)
            def _():
                dma(i + 1, h).start()

        for p in range(3):
            for v in range(NV):
                out_v[i, pl.ds(p * F + v * 16, 16)] = accs[p][v]
        return carry

    lax.fori_loop(0, EPW, ex_body, 0)
    pltpu.sync_copy(out_v, out_hbm.at[pl.ds(base, EPW)])


def kernel(conv_output, e1, e2):
    ee = jnp.concatenate(
        [e1.astype(jnp.int32), e2.astype(jnp.int32)], axis=1
    )
    ee = jnp.pad(ee, ((0, 0), (0, 14)))
    return _sc_piecewise_max(conv_output, ee)
